# baseline (speedup 1.0000x reference)
import functools

import jax
import jax.numpy as jnp
from jax import lax
from jax.experimental import pallas as pl
from jax.experimental.pallas import tpu as pltpu
from jax.experimental.pallas import tpu_sc as plsc

H = 128
G = 4
EB = 2560


def _mm(a, b):
    return jnp.dot(a, b, preferred_element_type=jnp.float32)


def _node_body(nb, xu, xm,
               encW1, encb1, encW2, encb2,
               impW1, impb1, impW2, impb2,
               intW1, intb1, intW2, intb2,
               npW, npb, Wproj,
               eeW2, eeb2, W1c, mlpb1,
               table_ref, wec_ref, btot_ref):
    relu = jax.nn.relu
    is_m = (pl.program_id(0) >= nb).astype(jnp.float32)
    x = xu[...] * (1.0 - is_m) + xm[...] * is_m

    h = _mm(relu(_mm(x, encW1[0]) + encb1[0]), encW2[0]) + encb2[0]
    imp = jax.nn.sigmoid(
        _mm(relu(_mm(h, impW1[...]) + impb1[...]), impW2[...]) + impb2[...])
    t = _mm(relu(_mm(h, intW1[...]) + intb1[...]), intW2[...]) + intb2[...]
    h = h + t * imp
    h = _mm(h, npW[...]) + npb[...]
    table_ref[...] = _mm(h, Wproj[0])

    wec_ref[...] = _mm(eeW2[...], W1c[...])
    btot_ref[...] = mlpb1[...] + _mm(eeb2[...], W1c[...])


def _edge_body(srcr, dstr, ea, eeW1, eeb1, wec, btot, W2, b2, W3, b3, out_ref):
    relu = jax.nn.relu
    e1 = relu(_mm(ea[...], eeW1[...]) + eeb1[...])
    z = relu(srcr[...] + dstr[...] + _mm(e1, wec[...]) + btot[...])
    h2 = relu(_mm(z, W2[...]) + b2[...])
    out_ref[...] = _mm(h2, W3[...]) + b3[...]


def _make_gather(b_grp, per_w, ch, nc, ns, n_half):
    n_ch = per_w // ch
    n_g = n_ch // 2
    b_half = b_grp // 2
    stage = (n_half // ns) // 8 * 8
    rem = n_half - stage * ns
    mesh = plsc.VectorSubcoreMesh(core_axis_name="c", subcore_axis_name="s")

    @functools.partial(
        pl.kernel,
        out_type=jax.ShapeDtypeStruct((b_grp, H), jnp.float32),
        mesh=mesh,
        scratch_types=[
            pltpu.VMEM((per_w,), jnp.int32),
            pltpu.VMEM((2, ch, H), jnp.float32),
            pltpu.VMEM_SHARED((n_half, H), jnp.float32),
            [pltpu.SemaphoreType.DMA] * 2,
            [pltpu.SemaphoreType.DMA] * 2,
        ],
    )
    def gather_k(table_hbm, idx_hbm, out_hbm, idx_v, rows_v, tab_s, g_sems, o_sems):
        c = lax.axis_index("c")
        s = lax.axis_index("s")
        base = pl.multiple_of(c * b_half + s * per_w, ch)

        pltpu.sync_copy(
            table_hbm.at[pl.ds(pl.multiple_of(c * n_half + s * stage, 8), stage)],
            tab_s.at[pl.ds(pl.multiple_of(s * stage, 8), stage)])
        if rem:
            @pl.when(s < rem // 8)
            def _():
                pltpu.sync_copy(
                    table_hbm.at[pl.ds(
                        pl.multiple_of(c * n_half + stage * ns + s * 8, 8), 8)],
                    tab_s.at[pl.ds(pl.multiple_of(stage * ns + s * 8, 8), 8)])
        pltpu.sync_copy(idx_hbm.at[pl.ds(base, per_w)], idx_v)
        plsc.subcore_barrier()

        def gather_copy(ci, slot):
            return pltpu.make_async_copy(
                tab_s.at[idx_v.at[pl.ds(pl.multiple_of(ci * ch, ch), ch)]],
                rows_v.at[slot], g_sems[slot])

        def out_copy(ci, slot):
            return pltpu.make_async_copy(
                rows_v.at[slot],
                out_hbm.at[pl.ds(pl.multiple_of(base + ci * ch, ch), ch)],
                o_sems[slot])

        def body(g, carry):
            for b in range(2):
                ci = g * 2 + b
                @pl.when(g >= 1)
                def _():
                    out_copy(ci - 2, b).wait()

                gather_copy(ci, b).start()

                if b == 1:
                    gather_copy(ci - 1, 0).wait()
                    out_copy(ci - 1, 0).start()
                else:
                    @pl.when(g >= 1)
                    def _():
                        gather_copy(ci - 1, 1).wait()
                        out_copy(ci - 1, 1).start()
            return carry

        lax.fori_loop(0, n_g, body, 0)

        gather_copy(n_ch - 1, 1).wait()
        out_copy(n_ch - 1, 1).start()
        for j in (n_ch - 2, n_ch - 1):
            out_copy(j, j % 2).wait()

    return gather_k


def kernel(x_user, x_merchant, edge_index, edge_index_rev, edge_attr, params):
    del edge_index_rev
    p = params
    n_u = x_user.shape[0]
    n_m = x_merchant.shape[0]
    n_edges = edge_index.shape[1]
    e_dim = edge_attr.shape[1]

    def row(v):
        return v.reshape(1, -1)

    W1a = p['mlp_W1'][:H]
    W1b = p['mlp_W1'][H:2 * H]
    W1c = p['mlp_W1'][2 * H:]

    nb = 5
    blk = n_u // nb

    def full(shape):
        return pl.BlockSpec(shape, lambda i: tuple(0 for _ in shape))

    def rel(shape):
        return pl.BlockSpec((1,) + shape, lambda i: (i // nb, 0, 0))

    table, wec, btot = pl.pallas_call(
        functools.partial(_node_body, nb),
        grid=(2 * nb,),
        in_specs=[pl.BlockSpec((blk, H), lambda i: (i % nb, 0)),
                  pl.BlockSpec((blk, H), lambda i: (i % nb, 0)),
                  rel((H, H)), rel((1, H)), rel((H, H)), rel((1, H)),
                  full((H, H // 2)), full((1, H // 2)), full((H // 2, 1)), full((1, 1)),
                  full((H, H)), full((1, H)), full((H, H)), full((1, H)),
                  full((H, H)), full((1, H)), rel((H, H)),
                  full((H, H)), full((1, H)), full((H, H)), full((1, H))],
        out_specs=[pl.BlockSpec((blk, H), lambda i: (i, 0)),
                   full((H, H)), full((1, H))],
        out_shape=[jax.ShapeDtypeStruct((n_u + n_m, H), jnp.float32),
                   jax.ShapeDtypeStruct((H, H), jnp.float32),
                   jax.ShapeDtypeStruct((1, H), jnp.float32)],
    )(x_user, x_merchant,
      jnp.stack([p['ue_W1'], p['me_W1']]), jnp.stack([row(p['ue_b1']), row(p['me_b1'])]),
      jnp.stack([p['ue_W2'], p['me_W2']]), jnp.stack([row(p['ue_b2']), row(p['me_b2'])]),
      p['imp_W1'], row(p['imp_b1']), p['imp_W2'], row(p['imp_b2']),
      p['int_W1'], row(p['int_b1']), p['int_W2'], row(p['int_b2']),
      p['np_W'], row(p['np_b']), jnp.stack([W1a, W1b]),
      p['ee_W2'], row(p['ee_b2']), W1c, row(p['mlp_b1']))

    info = plsc.get_sparse_core_info()
    nc, ns = info.num_cores, info.num_subcores
    ch = 64
    per_w = -(-n_edges // (G * ns * 640)) * 640
    q = per_w * ns
    b_pad = q * G
    nblk = q // EB

    srcp = jnp.pad(edge_index[0].astype(jnp.int32), (0, b_pad - n_edges))
    dstp = jnp.pad(edge_index[1].astype(jnp.int32), (0, b_pad - n_edges))
    ea_pad = jnp.pad(edge_attr, ((0, b_pad - n_edges), (0, 0)))

    gather_k = _make_gather(2 * q, per_w, ch, nc, ns, n_u)

    rows = []
    for g in range(G):
        idx_g = jnp.concatenate([srcp[g * q:(g + 1) * q], dstp[g * q:(g + 1) * q]])
        rows.append(gather_k(table, idx_g))

    logits_parts = []
    for g in range(G):
        rows_g = rows[g]
        logits_g = pl.pallas_call(
            _edge_body,
            grid=(nblk,),
            in_specs=[pl.BlockSpec((EB, H), lambda i: (i, 0)),
                      pl.BlockSpec((EB, H), lambda i: (i + nblk, 0)),
                      pl.BlockSpec((EB, e_dim), lambda i, g=g: (g * nblk + i, 0)),
                      full((e_dim, H)), full((1, H)),
                      full((H, H)), full((1, H)),
                      full((H, H // 2)), full((1, H // 2)),
                      full((H // 2, 2)), full((1, 2))],
            out_specs=pl.BlockSpec((EB, 2), lambda i: (i, 0)),
            out_shape=jax.ShapeDtypeStruct((q, 2), jnp.float32),
        )(rows_g, rows_g, ea_pad,
          p['ee_W1'], row(p['ee_b1']), wec, btot,
          p['mlp_W2'], row(p['mlp_b2']), p['mlp_W3'], row(p['mlp_b3']))
        logits_parts.append(logits_g)

    return jnp.concatenate(logits_parts)[:n_edges]


# speedup vs baseline: 1.1254x; 1.1254x over previous
import functools

import jax
import jax.numpy as jnp
from jax import lax
from jax.experimental import pallas as pl
from jax.experimental.pallas import tpu as pltpu
from jax.experimental.pallas import tpu_sc as plsc

H = 128
G = 4
EB = 2560


def _mm(a, b):
    return jnp.dot(a, b, preferred_element_type=jnp.float32)


def _node_body(nb, xu, xm,
               encW1, encb1, encW2, encb2,
               impW1, impb1, impW2, impb2,
               intW1, intb1, intW2, intb2,
               npW, npb, Wproj,
               eeW2, eeb2, W1c, mlpb1,
               table_ref, wec_ref, btot_ref):
    relu = jax.nn.relu
    is_m = (pl.program_id(0) >= nb).astype(jnp.float32)
    x = xu[...] * (1.0 - is_m) + xm[...] * is_m

    h = _mm(relu(_mm(x, encW1[0]) + encb1[0]), encW2[0]) + encb2[0]
    imp = jax.nn.sigmoid(
        _mm(relu(_mm(h, impW1[...]) + impb1[...]), impW2[...]) + impb2[...])
    t = _mm(relu(_mm(h, intW1[...]) + intb1[...]), intW2[...]) + intb2[...]
    h = h + t * imp
    h = _mm(h, npW[...]) + npb[...]
    table_ref[...] = _mm(h, Wproj[0])

    wec_ref[...] = _mm(eeW2[...], W1c[...])
    btot_ref[...] = mlpb1[...] + _mm(eeb2[...], W1c[...])


def _edge_body(srcr, dstr, ea, eeW1, eeb1, wec, btot, W2, b2, W3, b3, out_ref):
    relu = jax.nn.relu
    e1 = relu(_mm(ea[...], eeW1[...]) + eeb1[...])
    z = relu(srcr[...] + dstr[...] + _mm(e1, wec[...]) + btot[...])
    h2 = relu(_mm(z, W2[...]) + b2[...])
    out_ref[...] = _mm(h2, W3[...]) + b3[...]


def _make_gather(b_grp, per_w, ch, nc, ns, n_half):
    n_ch = per_w // ch
    n_g = n_ch // 2
    b_half = b_grp // 2
    stage = (n_half // ns) // 8 * 8
    rem = n_half - stage * ns
    mesh = plsc.VectorSubcoreMesh(core_axis_name="c", subcore_axis_name="s")

    @functools.partial(
        pl.kernel,
        out_type=jax.ShapeDtypeStruct((b_grp, H), jnp.float32),
        mesh=mesh,
        scratch_types=[
            pltpu.VMEM((per_w,), jnp.int32),
            pltpu.VMEM((2, ch, H), jnp.float32),
            pltpu.VMEM_SHARED((n_half, H), jnp.float32),
            [pltpu.SemaphoreType.DMA] * 2,
            [pltpu.SemaphoreType.DMA] * 2,
        ],
    )
    def gather_k(table_hbm, idx_hbm, out_hbm, idx_v, rows_v, tab_s, g_sems, o_sems):
        c = lax.axis_index("c")
        s = lax.axis_index("s")
        base = pl.multiple_of(c * b_half + s * per_w, ch)

        pltpu.sync_copy(
            table_hbm.at[pl.ds(pl.multiple_of(c * n_half + s * stage, 8), stage)],
            tab_s.at[pl.ds(pl.multiple_of(s * stage, 8), stage)])
        if rem:
            @pl.when(s < rem // 8)
            def _():
                pltpu.sync_copy(
                    table_hbm.at[pl.ds(
                        pl.multiple_of(c * n_half + stage * ns + s * 8, 8), 8)],
                    tab_s.at[pl.ds(pl.multiple_of(stage * ns + s * 8, 8), 8)])
        pltpu.sync_copy(idx_hbm.at[pl.ds(base, per_w)], idx_v)
        plsc.subcore_barrier()

        def gather_copy(ci, slot):
            return pltpu.make_async_copy(
                tab_s.at[idx_v.at[pl.ds(pl.multiple_of(ci * ch, ch), ch)]],
                rows_v.at[slot], g_sems[slot])

        def out_copy(ci, slot):
            return pltpu.make_async_copy(
                rows_v.at[slot],
                out_hbm.at[pl.ds(pl.multiple_of(base + ci * ch, ch), ch)],
                o_sems[slot])

        def body(g, carry):
            for b in range(2):
                ci = g * 2 + b
                @pl.when(g >= 1)
                def _():
                    out_copy(ci - 2, b).wait()

                gather_copy(ci, b).start()

                if b == 1:
                    gather_copy(ci - 1, 0).wait()
                    out_copy(ci - 1, 0).start()
                else:
                    @pl.when(g >= 1)
                    def _():
                        gather_copy(ci - 1, 1).wait()
                        out_copy(ci - 1, 1).start()
            return carry

        lax.fori_loop(0, n_g, body, 0)

        gather_copy(n_ch - 1, 1).wait()
        out_copy(n_ch - 1, 1).start()
        for j in (n_ch - 2, n_ch - 1):
            out_copy(j, j % 2).wait()

    return gather_k


def kernel(x_user, x_merchant, edge_index, edge_index_rev, edge_attr, params):
    del edge_index_rev
    p = params
    n_u = x_user.shape[0]
    n_m = x_merchant.shape[0]
    n_edges = edge_index.shape[1]
    e_dim = edge_attr.shape[1]

    def row(v):
        return v.reshape(1, -1)

    W1a = p['mlp_W1'][:H]
    W1b = p['mlp_W1'][H:2 * H]
    W1c = p['mlp_W1'][2 * H:]

    nb = 5
    blk = n_u // nb

    def full(shape):
        return pl.BlockSpec(shape, lambda i: tuple(0 for _ in shape))

    def rel(shape):
        return pl.BlockSpec((1,) + shape, lambda i: (i // nb, 0, 0))

    table, wec, btot = pl.pallas_call(
        functools.partial(_node_body, nb),
        grid=(2 * nb,),
        in_specs=[pl.BlockSpec((blk, H), lambda i: (i % nb, 0)),
                  pl.BlockSpec((blk, H), lambda i: (i % nb, 0)),
                  rel((H, H)), rel((1, H)), rel((H, H)), rel((1, H)),
                  full((H, H // 2)), full((1, H // 2)), full((H // 2, 1)), full((1, 1)),
                  full((H, H)), full((1, H)), full((H, H)), full((1, H)),
                  full((H, H)), full((1, H)), rel((H, H)),
                  full((H, H)), full((1, H)), full((H, H)), full((1, H))],
        out_specs=[pl.BlockSpec((blk, H), lambda i: (i, 0)),
                   full((H, H)), full((1, H))],
        out_shape=[jax.ShapeDtypeStruct((n_u + n_m, H), jnp.float32),
                   jax.ShapeDtypeStruct((H, H), jnp.float32),
                   jax.ShapeDtypeStruct((1, H), jnp.float32)],
    )(x_user, x_merchant,
      jnp.stack([p['ue_W1'], p['me_W1']]), jnp.stack([row(p['ue_b1']), row(p['me_b1'])]),
      jnp.stack([p['ue_W2'], p['me_W2']]), jnp.stack([row(p['ue_b2']), row(p['me_b2'])]),
      p['imp_W1'], row(p['imp_b1']), p['imp_W2'], row(p['imp_b2']),
      p['int_W1'], row(p['int_b1']), p['int_W2'], row(p['int_b2']),
      p['np_W'], row(p['np_b']), jnp.stack([W1a, W1b]),
      p['ee_W2'], row(p['ee_b2']), W1c, row(p['mlp_b1']))

    info = plsc.get_sparse_core_info()
    nc, ns = info.num_cores, info.num_subcores
    ch = 64
    per_w = -(-n_edges // (ns * 640)) * 640
    b_pad = per_w * ns
    b_all = 2 * b_pad

    pad = b_pad - n_edges
    idx_all = jnp.concatenate([
        jnp.pad(edge_index[0].astype(jnp.int32), (0, pad)),
        jnp.pad(edge_index[1].astype(jnp.int32), (0, pad)),
    ])

    gather_k = _make_gather(b_all, per_w, ch, nc, ns, n_u)
    rows = gather_k(table, idx_all)

    n_eb = n_edges // EB
    dst_off = b_pad // EB

    logits = pl.pallas_call(
        _edge_body,
        grid=(n_eb,),
        in_specs=[pl.BlockSpec((EB, H), lambda i: (i, 0)),
                  pl.BlockSpec((EB, H), lambda i: (i + dst_off, 0)),
                  pl.BlockSpec((EB, e_dim), lambda i: (i, 0)),
                  full((e_dim, H)), full((1, H)),
                  full((H, H)), full((1, H)),
                  full((H, H // 2)), full((1, H // 2)),
                  full((H // 2, 2)), full((1, 2))],
        out_specs=pl.BlockSpec((EB, 2), lambda i: (i, 0)),
        out_shape=jax.ShapeDtypeStruct((n_edges, 2), jnp.float32),
    )(rows, rows, edge_attr,
      p['ee_W1'], row(p['ee_b1']), wec, btot,
      p['mlp_W2'], row(p['mlp_b2']), p['mlp_W3'], row(p['mlp_b3']))

    return logits


# speedup vs baseline: 1.1295x; 1.0037x over previous
import functools

import jax
import jax.numpy as jnp
from jax import lax
from jax.experimental import pallas as pl
from jax.experimental.pallas import tpu as pltpu
from jax.experimental.pallas import tpu_sc as plsc

H = 128
EB = 2560


def _mm(a, b):
    return jnp.dot(a, b, preferred_element_type=jnp.float32)


def _node_body(nb, xu, xm,
               encW1, encb1, encW2, encb2,
               impW1, impb1, impW2, impb2,
               intW1, intb1, intW2, intb2,
               npW, npb, Wproj,
               eeW2, eeb2, W1c, mlpb1,
               table_ref, wec_ref, btot_ref):
    relu = jax.nn.relu
    is_m = (pl.program_id(0) >= nb).astype(jnp.float32)
    x = xu[...] * (1.0 - is_m) + xm[...] * is_m

    h = _mm(relu(_mm(x, encW1[0]) + encb1[0]), encW2[0]) + encb2[0]
    imp = jax.nn.sigmoid(
        _mm(relu(_mm(h, impW1[...]) + impb1[...]), impW2[...]) + impb2[...])
    t = _mm(relu(_mm(h, intW1[...]) + intb1[...]), intW2[...]) + intb2[...]
    h = h + t * imp
    h = _mm(h, npW[...]) + npb[...]
    table_ref[...] = _mm(h, Wproj[0])

    wec_ref[...] = _mm(eeW2[...], W1c[...])
    btot_ref[...] = mlpb1[...] + _mm(eeb2[...], W1c[...])


def _edge_body(srcr, dstr, ea, eeW1, eeb1, wec, btot, W2, b2, W3, b3, out_ref):
    relu = jax.nn.relu
    e1 = relu(_mm(ea[...], eeW1[...]) + eeb1[...])
    z = relu(srcr[...] + dstr[...] + _mm(e1, wec[...]) + btot[...])
    h2 = relu(_mm(z, W2[...]) + b2[...])
    out_ref[...] = _mm(h2, W3[...]) + b3[...]


def _make_gather(b_grp, per_w, ch, nc, ns, n_half):
    n_ch = per_w // ch
    n_g = n_ch // 2
    b_half = b_grp // 2
    stage = (n_half // ns) // 8 * 8
    rem = n_half - stage * ns
    mesh = plsc.VectorSubcoreMesh(core_axis_name="c", subcore_axis_name="s")

    @functools.partial(
        pl.kernel,
        out_type=jax.ShapeDtypeStruct((b_grp, H), jnp.float32),
        mesh=mesh,
        scratch_types=[
            pltpu.VMEM((per_w,), jnp.int32),
            pltpu.VMEM((2, ch, H), jnp.float32),
            pltpu.VMEM_SHARED((n_half, H), jnp.float32),
            [pltpu.SemaphoreType.DMA] * 2,
            [pltpu.SemaphoreType.DMA] * 2,
        ],
    )
    def gather_k(table_hbm, idx_hbm, out_hbm, idx_v, rows_v, tab_s, g_sems, o_sems):
        c = lax.axis_index("c")
        s = lax.axis_index("s")
        base = pl.multiple_of(c * b_half + s * per_w, ch)

        pltpu.sync_copy(
            table_hbm.at[pl.ds(pl.multiple_of(c * n_half + s * stage, 8), stage)],
            tab_s.at[pl.ds(pl.multiple_of(s * stage, 8), stage)])
        if rem:
            @pl.when(s < rem // 8)
            def _():
                pltpu.sync_copy(
                    table_hbm.at[pl.ds(
                        pl.multiple_of(c * n_half + stage * ns + s * 8, 8), 8)],
                    tab_s.at[pl.ds(pl.multiple_of(stage * ns + s * 8, 8), 8)])
        pltpu.sync_copy(idx_hbm.at[pl.ds(base, per_w)], idx_v)
        plsc.subcore_barrier()

        def gather_copy(ci, slot):
            return pltpu.make_async_copy(
                tab_s.at[idx_v.at[pl.ds(pl.multiple_of(ci * ch, ch), ch)]],
                rows_v.at[slot], g_sems[slot])

        def out_copy(ci, slot):
            return pltpu.make_async_copy(
                rows_v.at[slot],
                out_hbm.at[pl.ds(pl.multiple_of(base + ci * ch, ch), ch)],
                o_sems[slot])

        def body(g, carry):
            for b in range(2):
                ci = g * 2 + b
                @pl.when(g >= 1)
                def _():
                    out_copy(ci - 2, b).wait()

                gather_copy(ci, b).start()

                if b == 1:
                    gather_copy(ci - 1, 0).wait()
                    out_copy(ci - 1, 0).start()
                else:
                    @pl.when(g >= 1)
                    def _():
                        gather_copy(ci - 1, 1).wait()
                        out_copy(ci - 1, 1).start()
            return carry

        lax.fori_loop(0, n_g, body, 0)

        gather_copy(n_ch - 1, 1).wait()
        out_copy(n_ch - 1, 1).start()
        for j in (n_ch - 2, n_ch - 1):
            out_copy(j, j % 2).wait()

    return gather_k


def kernel(x_user, x_merchant, edge_index, edge_index_rev, edge_attr, params):
    del edge_index_rev
    p = params
    n_u = x_user.shape[0]
    n_m = x_merchant.shape[0]
    n_edges = edge_index.shape[1]
    e_dim = edge_attr.shape[1]

    def row(v):
        return v.reshape(1, -1)

    W1a = p['mlp_W1'][:H]
    W1b = p['mlp_W1'][H:2 * H]
    W1c = p['mlp_W1'][2 * H:]

    nb = 5
    blk = n_u // nb

    def full(shape):
        return pl.BlockSpec(shape, lambda i: tuple(0 for _ in shape))

    def rel(shape):
        return pl.BlockSpec((1,) + shape, lambda i: (i // nb, 0, 0))

    table, wec, btot = pl.pallas_call(
        functools.partial(_node_body, nb),
        grid=(2 * nb,),
        in_specs=[pl.BlockSpec((blk, H), lambda i: (i % nb, 0)),
                  pl.BlockSpec((blk, H), lambda i: (i % nb, 0)),
                  rel((H, H)), rel((1, H)), rel((H, H)), rel((1, H)),
                  full((H, H // 2)), full((1, H // 2)), full((H // 2, 1)), full((1, 1)),
                  full((H, H)), full((1, H)), full((H, H)), full((1, H)),
                  full((H, H)), full((1, H)), rel((H, H)),
                  full((H, H)), full((1, H)), full((H, H)), full((1, H))],
        out_specs=[pl.BlockSpec((blk, H), lambda i: (i, 0)),
                   full((H, H)), full((1, H))],
        out_shape=[jax.ShapeDtypeStruct((n_u + n_m, H), jnp.float32),
                   jax.ShapeDtypeStruct((H, H), jnp.float32),
                   jax.ShapeDtypeStruct((1, H), jnp.float32)],
    )(x_user, x_merchant,
      jnp.stack([p['ue_W1'], p['me_W1']]), jnp.stack([row(p['ue_b1']), row(p['me_b1'])]),
      jnp.stack([p['ue_W2'], p['me_W2']]), jnp.stack([row(p['ue_b2']), row(p['me_b2'])]),
      p['imp_W1'], row(p['imp_b1']), p['imp_W2'], row(p['imp_b2']),
      p['int_W1'], row(p['int_b1']), p['int_W2'], row(p['int_b2']),
      p['np_W'], row(p['np_b']), jnp.stack([W1a, W1b]),
      p['ee_W2'], row(p['ee_b2']), W1c, row(p['mlp_b1']))

    info = plsc.get_sparse_core_info()
    nc, ns = info.num_cores, info.num_subcores
    ch = 64
    per_w = -(-n_edges // (ns * 640)) * 640
    b_pad = per_w * ns
    b_all = 2 * b_pad

    pad = b_pad - n_edges
    idx_all = jnp.concatenate([
        jnp.pad(edge_index[0].astype(jnp.int32), (0, pad)),
        jnp.pad(edge_index[1].astype(jnp.int32), (0, pad)),
    ])

    gather_k = _make_gather(b_all, per_w, ch, nc, ns, n_u)
    rows = gather_k(table, idx_all)

    n_eb = n_edges // EB
    dst_off = b_pad // EB

    logits = pl.pallas_call(
        _edge_body,
        grid=(n_eb,),
        in_specs=[pl.BlockSpec((EB, H), lambda i: (i, 0)),
                  pl.BlockSpec((EB, H), lambda i: (i + dst_off, 0)),
                  pl.BlockSpec((EB, e_dim), lambda i: (i, 0)),
                  full((e_dim, H)), full((1, H)),
                  full((H, H)), full((1, H)),
                  full((H, H // 2)), full((1, H // 2)),
                  full((H // 2, 2)), full((1, 2))],
        out_specs=pl.BlockSpec((EB, 2), lambda i: (i, 0)),
        out_shape=jax.ShapeDtypeStruct((n_edges, 2), jnp.float32),
    )(rows, rows, edge_attr,
      p['ee_W1'], row(p['ee_b1']), wec, btot,
      p['mlp_W2'], row(p['mlp_b2']), p['mlp_W3'], row(p['mlp_b3']))

    return logits
